# baseline (speedup 1.0000x reference)
import functools

import jax
import jax.numpy as jnp
from jax import lax
from jax.experimental import pallas as pl
from jax.experimental.pallas import tpu as pltpu
from jax.experimental.pallas import tpu_sc as plsc

_NC = 2
_NS = 16
_NW = _NC * _NS

_CH = 16
_NBUF = 2


def _embed_call(n_per_w, nchunk, d, ids, weight):
  n = _NW * n_per_w
  mesh = plsc.VectorSubcoreMesh(core_axis_name="c", subcore_axis_name="s")

  @functools.partial(
      pl.kernel,
      out_type=jax.ShapeDtypeStruct((n, d), jnp.float32),
      mesh=mesh,
      scratch_types=[
          pltpu.VMEM((nchunk, _CH), jnp.int32),
          pltpu.VMEM((_NBUF, _CH, d), jnp.float32),
          pltpu.SemaphoreType.DMA,
          pltpu.SemaphoreType.DMA,
      ],
      compiler_params=pltpu.CompilerParams(use_tc_tiling_on_sc=False),
  )
  def body(ids_hbm, table_hbm, out_hbm, idx_v, rows_v, sem0, sem1):
    sems = (sem0, sem1)
    wid = lax.axis_index("s") * _NC + lax.axis_index("c")
    base = wid * n_per_w

    pltpu.sync_copy(ids_hbm.at[wid], idx_v)

    def start(c, b):
      pltpu.async_copy(table_hbm.at[idx_v.at[c]], rows_v.at[b], sems[b])

    def finish(c, b):
      pltpu.make_async_copy(
          table_hbm.at[pl.ds(0, _CH)], rows_v.at[b], sems[b]
      ).wait()
      pltpu.sync_copy(rows_v.at[b], out_hbm.at[pl.ds(base + c * _CH, _CH)])

    for b in range(_NBUF):
      start(b, b)

    def loop(i, carry):
      g = i * _NBUF
      for b in range(_NBUF):
        finish(g + b, b)
        start(g + b + _NBUF, b)
      return carry

    lax.fori_loop(0, (nchunk - _NBUF) // _NBUF, loop, 0)

    for b in range(_NBUF):
      finish(nchunk - _NBUF + b, b)

  return body(ids, weight)


def kernel(input_ids, weight):
  b, s = input_ids.shape
  v, d = weight.shape
  n = b * s
  assert n % (_NW * _CH) == 0
  n_per_w = n // _NW
  nchunk = n_per_w // _CH
  ids = input_ids.astype(jnp.int32).reshape(_NW, nchunk, _CH)
  out = _embed_call(n_per_w, nchunk, d, ids, weight)
  return out.reshape(b, s, d)


# speedup vs baseline: 1.9006x; 1.9006x over previous
import functools

import jax
import jax.numpy as jnp
from jax import lax
from jax.experimental import pallas as pl
from jax.experimental.pallas import tpu as pltpu
from jax.experimental.pallas import tpu_sc as plsc

_NC = 2
_NS = 16
_NW = _NC * _NS

_CH = 16
_NBUF = 2
_DMAIN = 2816
_DTAIL = 64


def _embed_call(n_per_w, nchunk, d, ids, weight, tail_table):
  n = _NW * n_per_w
  mesh = plsc.VectorSubcoreMesh(core_axis_name="c", subcore_axis_name="s")

  @functools.partial(
      pl.kernel,
      out_type=(
          jax.ShapeDtypeStruct((n, d), jnp.float32),
          jax.ShapeDtypeStruct((n, 128), jnp.float32),
      ),
      mesh=mesh,
      scratch_types=[
          pltpu.VMEM((nchunk, _CH), jnp.int32),
          pltpu.VMEM((_NBUF, _CH, _DMAIN), jnp.float32),
          pltpu.VMEM((_NBUF, _CH, 128), jnp.float32),
          pltpu.SemaphoreType.DMA,
          pltpu.SemaphoreType.DMA,
      ],
  )
  def body(ids_hbm, table_hbm, tail_hbm, out_hbm, tailout_hbm, idx_v, rows_v,
           tail_v, sem0, sem1):
    sems = (sem0, sem1)
    wid = lax.axis_index("s") * _NC + lax.axis_index("c")
    base = wid * n_per_w

    pltpu.sync_copy(ids_hbm.at[wid], idx_v)

    def start(c, b):
      pltpu.async_copy(
          table_hbm.at[idx_v.at[c], pl.ds(0, _DMAIN)], rows_v.at[b], sems[b]
      )
      pltpu.async_copy(tail_hbm.at[idx_v.at[c]], tail_v.at[b], sems[b])

    def finish(c, b):
      pltpu.make_async_copy(
          table_hbm.at[pl.ds(0, _CH), pl.ds(0, _DMAIN)], rows_v.at[b], sems[b]
      ).wait()
      pltpu.make_async_copy(
          tail_hbm.at[pl.ds(0, _CH)], tail_v.at[b], sems[b]
      ).wait()
      row0 = base + c * _CH
      pltpu.sync_copy(
          rows_v.at[b], out_hbm.at[pl.ds(row0, _CH), pl.ds(0, _DMAIN)]
      )
      pltpu.sync_copy(tail_v.at[b], tailout_hbm.at[pl.ds(row0, _CH)])

    for b in range(_NBUF):
      start(b, b)

    def loop(i, carry):
      g = i * _NBUF
      for b in range(_NBUF):
        finish(g + b, b)
        start(g + b + _NBUF, b)
      return carry

    lax.fori_loop(0, (nchunk - _NBUF) // _NBUF, loop, 0)

    for b in range(_NBUF):
      finish(nchunk - _NBUF + b, b)

  return body(ids, weight, tail_table)


def kernel(input_ids, weight):
  b, s = input_ids.shape
  v, d = weight.shape
  n = b * s
  assert n % (_NW * _CH) == 0 and d == _DMAIN + _DTAIL
  n_per_w = n // _NW
  nchunk = n_per_w // _CH
  ids = input_ids.astype(jnp.int32).reshape(_NW, nchunk, _CH)
  tail_table = jnp.pad(weight[:, _DMAIN:], ((0, 0), (0, 128 - _DTAIL)))
  out, tail_out = _embed_call(n_per_w, nchunk, d, ids, weight, tail_table)
  out = lax.dynamic_update_slice(out, tail_out[:, :_DTAIL], (0, _DMAIN))
  return out.reshape(b, s, d)


# speedup vs baseline: 3.4488x; 1.8146x over previous
import functools

import jax
import jax.numpy as jnp
from jax import lax
from jax.experimental import pallas as pl
from jax.experimental.pallas import tpu as pltpu
from jax.experimental.pallas import tpu_sc as plsc

_NC = 2
_NS = 16
_NW = _NC * _NS

_V = 201088
_H = 2880
_N = 16384
_HPW = _H // _NW
_NPAIR = _HPW // 2

_CHUNK = 12544
_NCHK = 16
_CSIZES = [_CHUNK] * (_NCHK - 1) + [_V - _CHUNK * (_NCHK - 1)]
_CSTARTS = [k * _CHUNK for k in range(_NCHK)]


def _embed_call(w_t, sid, perm, offs, nbatch, seq):
  mesh = plsc.VectorSubcoreMesh(core_axis_name="c", subcore_axis_name="s")
  cmax = max(_CSIZES)
  nrow = 2 * nbatch

  @functools.partial(
      pl.kernel,
      out_type=jax.ShapeDtypeStruct((nbatch * _H, seq), jnp.float32),
      mesh=mesh,
      scratch_types=[
          pltpu.VMEM((16384,), jnp.int32),
          pltpu.VMEM((16384,), jnp.int32),
          pltpu.VMEM((32,), jnp.int32),
          pltpu.VMEM((2, cmax), jnp.float32),
          pltpu.VMEM((2, cmax), jnp.float32),
          pltpu.VMEM((2, nbatch * seq), jnp.float32),
          pltpu.SemaphoreType.DMA,
          pltpu.SemaphoreType.DMA,
          pltpu.SemaphoreType.DMA,
      ],
      compiler_params=pltpu.CompilerParams(needs_layout_passes=False),
  )
  def body(wt_hbm, sid_hbm, perm_hbm, offs_hbm, out_hbm,
           sid_v, perm_v, offs_v, buf_a, buf_b, rows, sem0, sem1, osem):
    sems = (sem0, sem1)
    rings = (buf_a, buf_b)
    wid = lax.axis_index("s") * _NC + lax.axis_index("c")
    h0 = wid * _HPW

    pltpu.sync_copy(sid_hbm, sid_v)
    pltpu.sync_copy(perm_hbm, perm_v)
    pltpu.sync_copy(offs_hbm, offs_v)
    off_lo = offs_v[pl.ds(0, 16)]
    off_hi = offs_v[pl.ds(16, 16)]

    def off(k):
      return off_lo[k] if k < 16 else off_hi[k - 16]

    def start(p, k):
      h = h0 + 2 * p
      half = (_CSIZES[k] // 256) * 128
      for q, w in ((0, half), (half, _CSIZES[k] - half)):
        pltpu.async_copy(
            wt_hbm.at[pl.ds(h, 2), pl.ds(_CSTARTS[k] + q, w)],
            rings[k % 2].at[pl.ds(0, 2), pl.ds(q, w)],
            sems[k % 2],
        )

    def wait(k):
      half = (_CSIZES[k] // 256) * 128
      for q, w in ((0, half), (half, _CSIZES[k] - half)):
        pltpu.make_async_copy(
            wt_hbm.at[pl.ds(0, 2), pl.ds(q, w)],
            rings[k % 2].at[pl.ds(0, 2), pl.ds(q, w)],
            sems[k % 2],
        ).wait()

    def drain_out():
      for _ in range(nrow):
        pltpu.make_async_copy(
            rows.at[pl.ds(0, 1), pl.ds(0, seq)],
            out_hbm.at[pl.ds(0, 1), pl.ds(0, seq)],
            osem,
        ).wait()

    def gather_chunk(k):
      c0 = _CSTARTS[k]
      csz = _CSIZES[k]
      v0 = off(k) >> 4
      v1 = (off(k + 1) + 15) >> 4

      def vbody(v, carry):
        tok = sid_v[pl.ds(v * 16, 16)]
        pos = perm_v[pl.ds(v * 16, 16)]
        loc = jnp.minimum(jnp.maximum(tok - c0, 0), csz - 1)
        mask = tok >= c0
        for j in range(2):
          brow = jnp.full((16,), j, jnp.int32)
          vals = plsc.load_gather(rings[k % 2], [brow, loc])
          if k == 0:
            plsc.store_scatter(rows, [brow, pos], vals)
          else:
            plsc.store_scatter(rows, [brow, pos], vals, mask=mask)
        return carry

      lax.fori_loop(v0, v1, vbody, 0)

    start(0, 0)

    def pair_body(p, carry):
      for k in range(_NCHK):
        wait(k)
        if k + 1 < _NCHK:
          start(p, k + 1)
        else:
          @pl.when(p + 1 < _NPAIR)
          def _():
            start(p + 1, 0)
        if k == 0:
          @pl.when(p > 0)
          def _():
            drain_out()
        gather_chunk(k)
      h = h0 + 2 * p
      for j in range(2):
        for b in range(nbatch):
          pltpu.async_copy(
              rows.at[pl.ds(j, 1), pl.ds(b * seq, seq)],
              out_hbm.at[pl.ds(b * _H + h + j, 1), pl.ds(0, seq)],
              osem,
          )
      return carry

    lax.fori_loop(0, _NPAIR, pair_body, 0)
    drain_out()

  return body(w_t, sid, perm, offs)


def kernel(input_ids, weight):
  nb, seq = input_ids.shape
  v, d = weight.shape
  assert v == _V and d == _H and nb * seq == _N and seq == 4096
  ids = input_ids.astype(jnp.int32).reshape(_N)
  perm = jnp.argsort(ids).astype(jnp.int32)
  sid = ids[perm]
  starts = jnp.array(_CSTARTS + [_V], dtype=jnp.int32)
  offs = jnp.searchsorted(sid, starts, side="left").astype(jnp.int32)
  offs = jnp.pad(offs, (0, 32 - offs.shape[0]))
  w_t = weight.T
  out_flat = _embed_call(w_t, sid, perm, offs, nb, seq)
  out_t = out_flat.reshape(nb, _H, seq)
  return jnp.transpose(out_t, (0, 2, 1))
